# baseline (speedup 1.0000x reference)
import functools

import jax
import jax.numpy as jnp
import numpy as np
from jax import lax
from jax.experimental import pallas as pl
from jax.experimental.pallas import tpu as pltpu
from jax.experimental.pallas import tpu_sc as plsc

N = 10000
E = 320000
D = 128
H = 64
EPS = 1e-6
NSC = 2
NTILES = 16
EPT = E // NTILES
K = 80
NCHUNK = EPT // K
IDXBLK = 50
NBLK = NCHUNK // IDXBLK
NPAD = 10240
ROWS_PT = NPAD // NTILES


_RB = 2000


def _project_body(h_ref, w_ref, b_ref, ah_ref, g1f_ref, g2f_ref, g1b_ref, g2b_ref):
    hb = h_ref[...].astype(jnp.bfloat16)
    outs = []
    for j in range(7):
        outs.append(
            jnp.dot(hb, w_ref[j], preferred_element_type=jnp.float32)
            + b_ref[j][None, :]
        )
    Ah, Bh, Ch, Dh, B2h, C2h, D2h = [o.astype(jnp.bfloat16) if i else o
                                     for i, o in enumerate(outs)]
    ah_ref[...] = Ah
    g1f_ref[0] = jnp.concatenate([Dh[:, :H], Bh[:, :H]], axis=1)
    g1f_ref[1] = jnp.concatenate([Dh[:, H:], Bh[:, H:]], axis=1)
    g2f_ref[0] = Ch[:, :H]
    g2f_ref[1] = Ch[:, H:]
    g1b_ref[0] = jnp.concatenate([D2h[:, :H], B2h[:, :H]], axis=1)
    g1b_ref[1] = jnp.concatenate([D2h[:, H:], B2h[:, H:]], axis=1)
    g2b_ref[0] = C2h[:, :H]
    g2b_ref[1] = C2h[:, H:]


def _project(h, w, b):
    nblk = N // _RB
    return pl.pallas_call(
        _project_body,
        grid=(nblk,),
        in_specs=[
            pl.BlockSpec((_RB, D), lambda i: (i, 0)),
            pl.BlockSpec((7, D, D), lambda i: (0, 0, 0)),
            pl.BlockSpec((7, D), lambda i: (0, 0)),
        ],
        out_specs=[
            pl.BlockSpec((_RB, D), lambda i: (i, 0)),
            pl.BlockSpec((NSC, _RB, D), lambda i: (0, i, 0)),
            pl.BlockSpec((NSC, _RB, H), lambda i: (0, i, 0)),
            pl.BlockSpec((NSC, _RB, D), lambda i: (0, i, 0)),
            pl.BlockSpec((NSC, _RB, H), lambda i: (0, i, 0)),
        ],
        out_shape=[
            jax.ShapeDtypeStruct((N, D), jnp.float32),
            jax.ShapeDtypeStruct((NSC, N, D), jnp.bfloat16),
            jax.ShapeDtypeStruct((NSC, N, H), jnp.bfloat16),
            jax.ShapeDtypeStruct((NSC, N, D), jnp.bfloat16),
            jax.ShapeDtypeStruct((NSC, N, H), jnp.bfloat16),
        ],
    )(h, w, b)


def _edge_dir(c, s, acc_sh, bufs, idxsets, g1_hbm, g2_hbm,
              idxp_hbm, zeros_hbm, out_hbm):
    pltpu.sync_copy(zeros_hbm.at[pl.ds(s * ROWS_PT, ROWS_PT)],
                    acc_sh.at[pl.ds(s * ROWS_PT, ROWS_PT)])
    plsc.subcore_barrier()
    idxp_v = idxsets

    def issue(jj, b):
        g1b, g2b, _, sg, sc_, _ = bufs[b]
        pltpu.async_copy(g1_hbm.at[idxp_v.at[jj, pl.ds(0, K)]], g1b, sg)
        pltpu.async_copy(g2_hbm.at[idxp_v.at[jj, pl.ds(K, K)]], g2b, sc_)

    def compute(jj, b):
        g1b, g2b, payb, sg, sc_, ss = bufs[b]
        @pl.when(jj >= 2)
        def _():
            pltpu.make_async_copy(payb,
                                  acc_sh.at[idxp_v.at[jj, pl.ds(2 * K, K)]],
                                  ss).wait()
        pltpu.make_async_copy(g1_hbm.at[idxp_v.at[jj, pl.ds(0, K)]], g1b,
                              sg).wait()
        pltpu.make_async_copy(g2_hbm.at[idxp_v.at[jj, pl.ds(K, K)]], g2b,
                              sc_).wait()

        @plsc.parallel_loop(0, K, unroll=4)
        def edge(e):
            for j in range(H // 32):
                dhp = plsc.unpack(g1b[e, pl.ds(32 * j, 32)],
                                  format=plsc.PackFormat.INTERLEAVED)
                bhp = plsc.unpack(g1b[e, pl.ds(H + 32 * j, 32)],
                                  format=plsc.PackFormat.INTERLEAVED)
                chp = plsc.unpack(g2b[e, pl.ds(32 * j, 32)],
                                  format=plsc.PackFormat.INTERLEAVED)
                for t in range(2):
                    q = 2 * j + t
                    gate = 1.0 / (1.0 + jnp.exp(-(chp[t] + dhp[t])))
                    payb[e, pl.ds(16 * q, 16)] = gate * bhp[t]
                    payb[e, pl.ds(H + 16 * q, 16)] = gate

        pltpu.async_copy(payb, acc_sh.at[idxp_v.at[jj, pl.ds(2 * K, K)]],
                         ss, add=True)

        @pl.when(jj + 2 < IDXBLK)
        def _():
            issue(jj + 2, b)

    def block(blk, carry):
        rg = ((c * NTILES + s) * NBLK + blk) * IDXBLK
        pltpu.sync_copy(idxp_hbm.at[pl.ds(rg, IDXBLK)], idxp_v)
        issue(0, 0)
        issue(1, 1)

        def pair(t, carry2):
            compute(2 * t, 0)
            compute(2 * t + 1, 1)
            return carry2

        lax.fori_loop(0, IDXBLK // 2, pair, 0)
        for b in range(2):
            _, _, payb, _, _, ss = bufs[b]
            pltpu.make_async_copy(
                payb, acc_sh.at[idxp_v.at[IDXBLK - 2 + b, pl.ds(2 * K, K)]],
                ss).wait()
        return carry

    lax.fori_loop(0, NBLK, block, 0)
    plsc.subcore_barrier()
    pltpu.sync_copy(acc_sh.at[pl.ds(s * ROWS_PT, ROWS_PT)],
                    out_hbm.at[pl.ds(c * NPAD + s * ROWS_PT, ROWS_PT)])


def _edge_body(g1_hbm, g2_hbm, idxp_hbm, zeros_hbm,
               out_hbm, acc_sh, idxp_v,
               g1_v0, g1_v1, g2_v0, g2_v1, pay_v0, pay_v1,
               sg0, sg1, sc0, sc1, ss0, ss1):
    c = lax.axis_index("c")
    s = lax.axis_index("s")
    bufs = [(g1_v0, g2_v0, pay_v0, sg0, sc0, ss0),
            (g1_v1, g2_v1, pay_v1, sg1, sc1, ss1)]
    _edge_dir(c, s, acc_sh, bufs, idxp_v, g1_hbm, g2_hbm,
              idxp_hbm, zeros_hbm, out_hbm)


@functools.cache
def _make_edge_pass():
    return pl.kernel(
        _edge_body,
        mesh=plsc.VectorSubcoreMesh(core_axis_name="c", subcore_axis_name="s"),
        out_type=jax.ShapeDtypeStruct((NSC * NPAD, D), jnp.float32),
        compiler_params=pltpu.CompilerParams(use_tc_tiling_on_sc=False, needs_layout_passes=False),
        scratch_types=[
            pltpu.VMEM_SHARED((NPAD, D), jnp.float32),
            pltpu.VMEM((IDXBLK, 3 * K), jnp.int32),
            pltpu.VMEM((K, D), jnp.bfloat16),
            pltpu.VMEM((K, D), jnp.bfloat16),
            pltpu.VMEM((K, H), jnp.bfloat16),
            pltpu.VMEM((K, H), jnp.bfloat16),
            pltpu.VMEM((K, D), jnp.float32),
            pltpu.VMEM((K, D), jnp.float32),
            pltpu.SemaphoreType.DMA,
            pltpu.SemaphoreType.DMA,
            pltpu.SemaphoreType.DMA,
            pltpu.SemaphoreType.DMA,
            pltpu.SemaphoreType.DMA,
            pltpu.SemaphoreType.DMA,
        ],
    )


def _edge_pass(*args):
    return _make_edge_pass()(*args)


def _combine_body(last, ah_ref, accf_ref, accb_ref, out_ref):
    ah = ah_ref[...]
    numf = jnp.concatenate([accf_ref[0][:, :H], accf_ref[1][:, :H]], axis=1)
    denf = jnp.concatenate([accf_ref[0][:, H:], accf_ref[1][:, H:]], axis=1)
    numb = jnp.concatenate([accb_ref[0][:, :H], accb_ref[1][:, :H]], axis=1)
    denb = jnp.concatenate([accb_ref[0][:, H:], accb_ref[1][:, H:]], axis=1)
    out = ah + numf / (denf + EPS) + numb / (denb + EPS)
    if last:
        m = jnp.max(out, axis=1, keepdims=True)
        sh = out - m
        out = sh - jnp.log(jnp.sum(jnp.exp(sh), axis=1, keepdims=True))
    else:
        out = jnp.maximum(out, 0.0)
    out_ref[...] = out


def _combine(ah, accf, accb, last):
    nblk = N // _RB
    return pl.pallas_call(
        functools.partial(_combine_body, last),
        grid=(nblk,),
        in_specs=[
            pl.BlockSpec((_RB, D), lambda i: (i, 0)),
            pl.BlockSpec((NSC, _RB, D), lambda i: (0, i, 0)),
            pl.BlockSpec((NSC, _RB, D), lambda i: (0, i, 0)),
        ],
        out_specs=pl.BlockSpec((_RB, D), lambda i: (i, 0)),
        out_shape=jax.ShapeDtypeStruct((N, D), jnp.float32),
    )(ah, accf, accb)


def kernel(h, edge_index, params):
    src = edge_index[0]
    dst = edge_index[1]
    src2 = jnp.concatenate([src, src + N]).reshape(-1, K)
    dst2 = jnp.concatenate([dst, dst + N]).reshape(-1, K)
    src1 = jnp.concatenate([src, src]).reshape(-1, K)
    dst1 = jnp.concatenate([dst, dst]).reshape(-1, K)
    idxpf = jnp.concatenate([src2, dst2, dst1], axis=1)
    idxpb = jnp.concatenate([dst2, src2, src1], axis=1)
    zeros = jnp.zeros((NPAD, D), jnp.float32)

    names = ["A", "B", "C", "D", "B2", "C2", "D2"]
    perm = np.empty((D,), np.int32)
    for jj in range(D // 32):
        for t in range(16):
            perm[32 * jj + 2 * t] = 32 * jj + t
            perm[32 * jj + 2 * t + 1] = 32 * jj + 16 + t
    perm = jnp.asarray(perm)
    for i, p in enumerate(params):
        w = jnp.stack([p["W_A"]] + [p["W_" + n][:, perm] for n in names[1:]]
                      ).astype(jnp.bfloat16)
        b = jnp.stack([p["b_A"]] + [p["b_" + n][perm] for n in names[1:]])
        ah, g1f, g2f, g1b, g2b = _project(h, w, b)
        accf = _edge_pass(g1f.reshape(NSC * N, D), g2f.reshape(NSC * N, H),
                          idxpf, zeros)
        accb = _edge_pass(g1b.reshape(NSC * N, D), g2b.reshape(NSC * N, H),
                          idxpb, zeros)
        h = _combine(ah, accf.reshape(NSC, NPAD, D), accb.reshape(NSC, NPAD, D),
                     last=(i == len(params) - 1))
    return h


# speedup vs baseline: 1.0100x; 1.0100x over previous
import functools

import jax
import jax.numpy as jnp
import numpy as np
from jax import lax
from jax.experimental import pallas as pl
from jax.experimental.pallas import tpu as pltpu
from jax.experimental.pallas import tpu_sc as plsc

N = 10000
E = 320000
D = 128
H = 64
EPS = 1e-6
NSC = 2
NTILES = 16
EPT = E // NTILES
K = 80
NCHUNK = EPT // K
IDXBLK = 50
NBLK = NCHUNK // IDXBLK
NPAD = 10240
ROWS_PT = NPAD // NTILES


_RB = 2000


def _project_body(h_ref, w_ref, b_ref, ah_ref, g1f_ref, g2f_ref, g1b_ref, g2b_ref):
    hb = h_ref[...]
    outs = []
    for j in range(7):
        outs.append(
            jnp.dot(hb, w_ref[j], preferred_element_type=jnp.float32)
            + b_ref[j][None, :]
        )
    Ah, Bh, Ch, Dh, B2h, C2h, D2h = [o.astype(jnp.bfloat16) if i else o
                                     for i, o in enumerate(outs)]
    ah_ref[...] = Ah
    g1f_ref[0] = jnp.concatenate([Dh[:, :H], Bh[:, :H]], axis=1)
    g1f_ref[1] = jnp.concatenate([Dh[:, H:], Bh[:, H:]], axis=1)
    g2f_ref[0] = Ch[:, :H]
    g2f_ref[1] = Ch[:, H:]
    g1b_ref[0] = jnp.concatenate([D2h[:, :H], B2h[:, :H]], axis=1)
    g1b_ref[1] = jnp.concatenate([D2h[:, H:], B2h[:, H:]], axis=1)
    g2b_ref[0] = C2h[:, :H]
    g2b_ref[1] = C2h[:, H:]


def _project(h, w, b):
    nblk = N // _RB
    return pl.pallas_call(
        _project_body,
        grid=(nblk,),
        in_specs=[
            pl.BlockSpec((_RB, D), lambda i: (i, 0)),
            pl.BlockSpec((7, D, D), lambda i: (0, 0, 0)),
            pl.BlockSpec((7, D), lambda i: (0, 0)),
        ],
        out_specs=[
            pl.BlockSpec((_RB, D), lambda i: (i, 0)),
            pl.BlockSpec((NSC, _RB, D), lambda i: (0, i, 0)),
            pl.BlockSpec((NSC, _RB, H), lambda i: (0, i, 0)),
            pl.BlockSpec((NSC, _RB, D), lambda i: (0, i, 0)),
            pl.BlockSpec((NSC, _RB, H), lambda i: (0, i, 0)),
        ],
        out_shape=[
            jax.ShapeDtypeStruct((N, D), jnp.float32),
            jax.ShapeDtypeStruct((NSC, N, D), jnp.bfloat16),
            jax.ShapeDtypeStruct((NSC, N, H), jnp.bfloat16),
            jax.ShapeDtypeStruct((NSC, N, D), jnp.bfloat16),
            jax.ShapeDtypeStruct((NSC, N, H), jnp.bfloat16),
        ],
    )(h, w, b)


def _edge_dir(c, s, acc_sh, bufs, idxsets, g1_hbm, g2_hbm,
              idxp_hbm, zeros_hbm, out_hbm):
    pltpu.sync_copy(zeros_hbm.at[pl.ds(s * ROWS_PT, ROWS_PT)],
                    acc_sh.at[pl.ds(s * ROWS_PT, ROWS_PT)])
    plsc.subcore_barrier()
    idxp_v = idxsets

    def issue(jj, b):
        g1b, g2b, _, sg, sc_, _ = bufs[b]
        pltpu.async_copy(g1_hbm.at[idxp_v.at[jj, pl.ds(0, K)]], g1b, sg)
        pltpu.async_copy(g2_hbm.at[idxp_v.at[jj, pl.ds(K, K)]], g2b, sc_)

    def compute(jj, b):
        g1b, g2b, payb, sg, sc_, ss = bufs[b]
        @pl.when(jj >= 2)
        def _():
            pltpu.make_async_copy(payb,
                                  acc_sh.at[idxp_v.at[jj, pl.ds(2 * K, K)]],
                                  ss).wait()
        pltpu.make_async_copy(g1_hbm.at[idxp_v.at[jj, pl.ds(0, K)]], g1b,
                              sg).wait()
        pltpu.make_async_copy(g2_hbm.at[idxp_v.at[jj, pl.ds(K, K)]], g2b,
                              sc_).wait()

        @plsc.parallel_loop(0, K, unroll=4)
        def edge(e):
            for j in range(H // 32):
                dhp = plsc.unpack(g1b[e, pl.ds(32 * j, 32)],
                                  format=plsc.PackFormat.INTERLEAVED)
                bhp = plsc.unpack(g1b[e, pl.ds(H + 32 * j, 32)],
                                  format=plsc.PackFormat.INTERLEAVED)
                chp = plsc.unpack(g2b[e, pl.ds(32 * j, 32)],
                                  format=plsc.PackFormat.INTERLEAVED)
                for t in range(2):
                    q = 2 * j + t
                    gate = 1.0 / (1.0 + jnp.exp(-(chp[t] + dhp[t])))
                    payb[e, pl.ds(16 * q, 16)] = gate * bhp[t]
                    payb[e, pl.ds(H + 16 * q, 16)] = gate

        pltpu.async_copy(payb, acc_sh.at[idxp_v.at[jj, pl.ds(2 * K, K)]],
                         ss, add=True)

        @pl.when(jj + 2 < IDXBLK)
        def _():
            issue(jj + 2, b)

    def block(blk, carry):
        rg = ((c * NTILES + s) * NBLK + blk) * IDXBLK
        pltpu.sync_copy(idxp_hbm.at[pl.ds(rg, IDXBLK)], idxp_v)
        issue(0, 0)
        issue(1, 1)

        def pair(t, carry2):
            compute(2 * t, 0)
            compute(2 * t + 1, 1)
            return carry2

        lax.fori_loop(0, IDXBLK // 2, pair, 0)
        for b in range(2):
            _, _, payb, _, _, ss = bufs[b]
            pltpu.make_async_copy(
                payb, acc_sh.at[idxp_v.at[IDXBLK - 2 + b, pl.ds(2 * K, K)]],
                ss).wait()
        return carry

    lax.fori_loop(0, NBLK, block, 0)
    plsc.subcore_barrier()
    pltpu.sync_copy(acc_sh.at[pl.ds(s * ROWS_PT, ROWS_PT)],
                    out_hbm.at[pl.ds(c * NPAD + s * ROWS_PT, ROWS_PT)])


def _edge_body(g1_hbm, g2_hbm, idxp_hbm, zeros_hbm,
               out_hbm, acc_sh, idxp_v,
               g1_v0, g1_v1, g2_v0, g2_v1, pay_v0, pay_v1,
               sg0, sg1, sc0, sc1, ss0, ss1):
    c = lax.axis_index("c")
    s = lax.axis_index("s")
    bufs = [(g1_v0, g2_v0, pay_v0, sg0, sc0, ss0),
            (g1_v1, g2_v1, pay_v1, sg1, sc1, ss1)]
    _edge_dir(c, s, acc_sh, bufs, idxp_v, g1_hbm, g2_hbm,
              idxp_hbm, zeros_hbm, out_hbm)


@functools.cache
def _make_edge_pass():
    return pl.kernel(
        _edge_body,
        mesh=plsc.VectorSubcoreMesh(core_axis_name="c", subcore_axis_name="s"),
        out_type=jax.ShapeDtypeStruct((NSC * NPAD, D), jnp.float32),
        compiler_params=pltpu.CompilerParams(use_tc_tiling_on_sc=False, needs_layout_passes=False),
        scratch_types=[
            pltpu.VMEM_SHARED((NPAD, D), jnp.float32),
            pltpu.VMEM((IDXBLK, 3 * K), jnp.int32),
            pltpu.VMEM((K, D), jnp.bfloat16),
            pltpu.VMEM((K, D), jnp.bfloat16),
            pltpu.VMEM((K, H), jnp.bfloat16),
            pltpu.VMEM((K, H), jnp.bfloat16),
            pltpu.VMEM((K, D), jnp.float32),
            pltpu.VMEM((K, D), jnp.float32),
            pltpu.SemaphoreType.DMA,
            pltpu.SemaphoreType.DMA,
            pltpu.SemaphoreType.DMA,
            pltpu.SemaphoreType.DMA,
            pltpu.SemaphoreType.DMA,
            pltpu.SemaphoreType.DMA,
        ],
    )


def _edge_pass(*args):
    return _make_edge_pass()(*args)


def _combine_body(last, ah_ref, accf_ref, accb_ref, out_ref):
    ah = ah_ref[...]
    numf = jnp.concatenate([accf_ref[0][:, :H], accf_ref[1][:, :H]], axis=1)
    denf = jnp.concatenate([accf_ref[0][:, H:], accf_ref[1][:, H:]], axis=1)
    numb = jnp.concatenate([accb_ref[0][:, :H], accb_ref[1][:, :H]], axis=1)
    denb = jnp.concatenate([accb_ref[0][:, H:], accb_ref[1][:, H:]], axis=1)
    out = ah + numf / (denf + EPS) + numb / (denb + EPS)
    if last:
        m = jnp.max(out, axis=1, keepdims=True)
        sh = out - m
        out = sh - jnp.log(jnp.sum(jnp.exp(sh), axis=1, keepdims=True))
    else:
        out = jnp.maximum(out, 0.0)
    out_ref[...] = out


def _combine(ah, accf, accb, last):
    nblk = N // _RB
    return pl.pallas_call(
        functools.partial(_combine_body, last),
        grid=(nblk,),
        in_specs=[
            pl.BlockSpec((_RB, D), lambda i: (i, 0)),
            pl.BlockSpec((NSC, _RB, D), lambda i: (0, i, 0)),
            pl.BlockSpec((NSC, _RB, D), lambda i: (0, i, 0)),
        ],
        out_specs=pl.BlockSpec((_RB, D), lambda i: (i, 0)),
        out_shape=jax.ShapeDtypeStruct((N, D), jnp.float32),
    )(ah, accf, accb)


def kernel(h, edge_index, params):
    src = edge_index[0]
    dst = edge_index[1]
    src2 = jnp.concatenate([src, src + N]).reshape(-1, K)
    dst2 = jnp.concatenate([dst, dst + N]).reshape(-1, K)
    src1 = jnp.concatenate([src, src]).reshape(-1, K)
    dst1 = jnp.concatenate([dst, dst]).reshape(-1, K)
    idxpf = jnp.concatenate([src2, dst2, dst1], axis=1)
    idxpb = jnp.concatenate([dst2, src2, src1], axis=1)
    zeros = jnp.zeros((NPAD, D), jnp.float32)

    names = ["A", "B", "C", "D", "B2", "C2", "D2"]
    perm = np.empty((D,), np.int32)
    for jj in range(D // 32):
        for t in range(16):
            perm[32 * jj + 2 * t] = 32 * jj + t
            perm[32 * jj + 2 * t + 1] = 32 * jj + 16 + t
    perm = jnp.asarray(perm)
    for i, p in enumerate(params):
        w = jnp.stack([p["W_A"]] + [p["W_" + n][:, perm] for n in names[1:]])
        b = jnp.stack([p["b_A"]] + [p["b_" + n][perm] for n in names[1:]])
        ah, g1f, g2f, g1b, g2b = _project(h, w, b)
        accf = _edge_pass(g1f.reshape(NSC * N, D), g2f.reshape(NSC * N, H),
                          idxpf, zeros)
        accb = _edge_pass(g1b.reshape(NSC * N, D), g2b.reshape(NSC * N, H),
                          idxpb, zeros)
        h = _combine(ah, accf.reshape(NSC, NPAD, D), accb.reshape(NSC, NPAD, D),
                     last=(i == len(params) - 1))
    return h


# speedup vs baseline: 1.0102x; 1.0002x over previous
import functools

import jax
import jax.numpy as jnp
import numpy as np
from jax import lax
from jax.experimental import pallas as pl
from jax.experimental.pallas import tpu as pltpu
from jax.experimental.pallas import tpu_sc as plsc

N = 10000
E = 320000
D = 128
H = 64
EPS = 1e-6
NSC = 2
NTILES = 16
EPT = E // NTILES
K = 80
NCHUNK = EPT // K
IDXBLK = 50
NBLK = NCHUNK // IDXBLK
NPAD = 10240
ROWS_PT = NPAD // NTILES


_RB = 2000


def _project_body(h_ref, w_ref, b_ref, ah_ref, g1f_ref, g2f_ref, g1b_ref, g2b_ref):
    hb = h_ref[...]
    outs = []
    for j in range(7):
        outs.append(
            jnp.dot(hb, w_ref[j], preferred_element_type=jnp.float32)
            + b_ref[j][None, :]
        )
    Ah, Bh, Ch, Dh, B2h, C2h, D2h = [o.astype(jnp.bfloat16) if i else o
                                     for i, o in enumerate(outs)]
    ah_ref[...] = Ah
    g1f_ref[0] = jnp.concatenate([Dh[:, :H], Bh[:, :H]], axis=1)
    g1f_ref[1] = jnp.concatenate([Dh[:, H:], Bh[:, H:]], axis=1)
    g2f_ref[0] = Ch[:, :H]
    g2f_ref[1] = Ch[:, H:]
    g1b_ref[0] = jnp.concatenate([D2h[:, :H], B2h[:, :H]], axis=1)
    g1b_ref[1] = jnp.concatenate([D2h[:, H:], B2h[:, H:]], axis=1)
    g2b_ref[0] = C2h[:, :H]
    g2b_ref[1] = C2h[:, H:]


def _project(h, w, b):
    nblk = N // _RB
    return pl.pallas_call(
        _project_body,
        grid=(nblk,),
        in_specs=[
            pl.BlockSpec((_RB, D), lambda i: (i, 0)),
            pl.BlockSpec((7, D, D), lambda i: (0, 0, 0)),
            pl.BlockSpec((7, D), lambda i: (0, 0)),
        ],
        out_specs=[
            pl.BlockSpec((_RB, D), lambda i: (i, 0)),
            pl.BlockSpec((NSC, _RB, D), lambda i: (0, i, 0)),
            pl.BlockSpec((NSC, _RB, H), lambda i: (0, i, 0)),
            pl.BlockSpec((NSC, _RB, D), lambda i: (0, i, 0)),
            pl.BlockSpec((NSC, _RB, H), lambda i: (0, i, 0)),
        ],
        out_shape=[
            jax.ShapeDtypeStruct((N, D), jnp.float32),
            jax.ShapeDtypeStruct((NSC, N, D), jnp.bfloat16),
            jax.ShapeDtypeStruct((NSC, N, H), jnp.bfloat16),
            jax.ShapeDtypeStruct((NSC, N, D), jnp.bfloat16),
            jax.ShapeDtypeStruct((NSC, N, H), jnp.bfloat16),
        ],
    )(h, w, b)


def _edge_dir(c, s, acc_sh, bufs, idxp_v, g1_hbm, g2_hbm,
              idxp_hbm, zeros_hbm, out_hbm):
    pltpu.sync_copy(zeros_hbm.at[pl.ds(s * ROWS_PT, ROWS_PT)],
                    acc_sh.at[pl.ds(s * ROWS_PT, ROWS_PT)])
    plsc.subcore_barrier()

    def issue(jj, b):
        g1b, g2b, _, sg, sc_, _ = bufs[b]
        pltpu.async_copy(g1_hbm.at[idxp_v.at[jj, pl.ds(0, K)]], g1b, sg)
        pltpu.async_copy(g2_hbm.at[idxp_v.at[jj, pl.ds(K, K)]], g2b, sc_)

    def compute(jj, b):
        g1b, g2b, payb, sg, sc_, ss = bufs[b]
        @pl.when(jj >= 2)
        def _():
            pltpu.make_async_copy(payb,
                                  acc_sh.at[idxp_v.at[jj, pl.ds(2 * K, K)]],
                                  ss).wait()
        pltpu.make_async_copy(g1_hbm.at[idxp_v.at[jj, pl.ds(0, K)]], g1b,
                              sg).wait()
        pltpu.make_async_copy(g2_hbm.at[idxp_v.at[jj, pl.ds(K, K)]], g2b,
                              sc_).wait()

        @plsc.parallel_loop(0, K, unroll=4)
        def edge(e):
            for j in range(H // 32):
                dhp = plsc.unpack(g1b[e, pl.ds(32 * j, 32)],
                                  format=plsc.PackFormat.INTERLEAVED)
                bhp = plsc.unpack(g1b[e, pl.ds(H + 32 * j, 32)],
                                  format=plsc.PackFormat.INTERLEAVED)
                chp = plsc.unpack(g2b[e, pl.ds(32 * j, 32)],
                                  format=plsc.PackFormat.INTERLEAVED)
                for t in range(2):
                    q = 2 * j + t
                    gate = 1.0 / (1.0 + jnp.exp(-(chp[t] + dhp[t])))
                    payb[e, pl.ds(16 * q, 16)] = gate * bhp[t]
                    payb[e, pl.ds(H + 16 * q, 16)] = gate

        pltpu.async_copy(payb, acc_sh.at[idxp_v.at[jj, pl.ds(2 * K, K)]],
                         ss, add=True)

        @pl.when(jj + 2 < IDXBLK)
        def _():
            issue(jj + 2, b)

    def block(blk, carry):
        rg = ((c * NTILES + s) * NBLK + blk) * IDXBLK
        pltpu.sync_copy(idxp_hbm.at[pl.ds(rg, IDXBLK)], idxp_v)
        issue(0, 0)
        issue(1, 1)

        def pair(t, carry2):
            compute(2 * t, 0)
            compute(2 * t + 1, 1)
            return carry2

        lax.fori_loop(0, IDXBLK // 2, pair, 0)
        for b in range(2):
            _, _, payb, _, _, ss = bufs[b]
            pltpu.make_async_copy(
                payb, acc_sh.at[idxp_v.at[IDXBLK - 2 + b, pl.ds(2 * K, K)]],
                ss).wait()
        return carry

    lax.fori_loop(0, NBLK, block, 0)
    plsc.subcore_barrier()
    pltpu.sync_copy(acc_sh.at[pl.ds(s * ROWS_PT, ROWS_PT)],
                    out_hbm.at[pl.ds(c * NPAD + s * ROWS_PT, ROWS_PT)])


def _edge_body(g1_hbm, g2_hbm, idxp_hbm, zeros_hbm,
               out_hbm, acc_sh, idxp_v,
               g1_v0, g1_v1, g2_v0, g2_v1, pay_v0, pay_v1,
               sg0, sg1, sc0, sc1, ss0, ss1):
    c = lax.axis_index("c")
    s = lax.axis_index("s")
    bufs = [(g1_v0, g2_v0, pay_v0, sg0, sc0, ss0),
            (g1_v1, g2_v1, pay_v1, sg1, sc1, ss1)]
    _edge_dir(c, s, acc_sh, bufs, idxp_v, g1_hbm, g2_hbm,
              idxp_hbm, zeros_hbm, out_hbm)


@functools.cache
def _make_edge_pass():
    return pl.kernel(
        _edge_body,
        mesh=plsc.VectorSubcoreMesh(core_axis_name="c", subcore_axis_name="s"),
        out_type=jax.ShapeDtypeStruct((NSC * NPAD, D), jnp.float32),
        compiler_params=pltpu.CompilerParams(use_tc_tiling_on_sc=False, needs_layout_passes=False),
        scratch_types=[
            pltpu.VMEM_SHARED((NPAD, D), jnp.float32),
            pltpu.VMEM((IDXBLK, 3 * K), jnp.int32),
            pltpu.VMEM((K, D), jnp.bfloat16),
            pltpu.VMEM((K, D), jnp.bfloat16),
            pltpu.VMEM((K, H), jnp.bfloat16),
            pltpu.VMEM((K, H), jnp.bfloat16),
            pltpu.VMEM((K, D), jnp.float32),
            pltpu.VMEM((K, D), jnp.float32),
            pltpu.SemaphoreType.DMA,
            pltpu.SemaphoreType.DMA,
            pltpu.SemaphoreType.DMA,
            pltpu.SemaphoreType.DMA,
            pltpu.SemaphoreType.DMA,
            pltpu.SemaphoreType.DMA,
        ],
    )


def _edge_pass(*args):
    return _make_edge_pass()(*args)


def _combine_body(last, ah_ref, accf_ref, accb_ref, out_ref):
    ah = ah_ref[...]
    numf = jnp.concatenate([accf_ref[0][:, :H], accf_ref[1][:, :H]], axis=1)
    denf = jnp.concatenate([accf_ref[0][:, H:], accf_ref[1][:, H:]], axis=1)
    numb = jnp.concatenate([accb_ref[0][:, :H], accb_ref[1][:, :H]], axis=1)
    denb = jnp.concatenate([accb_ref[0][:, H:], accb_ref[1][:, H:]], axis=1)
    out = ah + numf / (denf + EPS) + numb / (denb + EPS)
    if last:
        m = jnp.max(out, axis=1, keepdims=True)
        sh = out - m
        out = sh - jnp.log(jnp.sum(jnp.exp(sh), axis=1, keepdims=True))
    else:
        out = jnp.maximum(out, 0.0)
    out_ref[...] = out


def _combine(ah, accf, accb, last):
    nblk = N // _RB
    return pl.pallas_call(
        functools.partial(_combine_body, last),
        grid=(nblk,),
        in_specs=[
            pl.BlockSpec((_RB, D), lambda i: (i, 0)),
            pl.BlockSpec((NSC, _RB, D), lambda i: (0, i, 0)),
            pl.BlockSpec((NSC, _RB, D), lambda i: (0, i, 0)),
        ],
        out_specs=pl.BlockSpec((_RB, D), lambda i: (i, 0)),
        out_shape=jax.ShapeDtypeStruct((N, D), jnp.float32),
    )(ah, accf, accb)


def kernel(h, edge_index, params):
    src = edge_index[0]
    dst = edge_index[1]
    src2 = jnp.concatenate([src, src + N]).reshape(-1, K)
    dst2 = jnp.concatenate([dst, dst + N]).reshape(-1, K)
    src1 = jnp.concatenate([src, src]).reshape(-1, K)
    dst1 = jnp.concatenate([dst, dst]).reshape(-1, K)
    idxpf = jnp.concatenate([src2, dst2, dst1], axis=1)
    idxpb = jnp.concatenate([dst2, src2, src1], axis=1)
    zeros = jnp.zeros((NPAD, D), jnp.float32)

    names = ["A", "B", "C", "D", "B2", "C2", "D2"]
    perm = np.empty((D,), np.int32)
    for jj in range(D // 32):
        for t in range(16):
            perm[32 * jj + 2 * t] = 32 * jj + t
            perm[32 * jj + 2 * t + 1] = 32 * jj + 16 + t
    perm = jnp.asarray(perm)
    for i, p in enumerate(params):
        w = jnp.stack([p["W_A"]] + [p["W_" + n][:, perm] for n in names[1:]])
        b = jnp.stack([p["b_A"]] + [p["b_" + n][perm] for n in names[1:]])
        ah, g1f, g2f, g1b, g2b = _project(h, w, b)
        accf = _edge_pass(g1f.reshape(NSC * N, D), g2f.reshape(NSC * N, H),
                          idxpf, zeros)
        accb = _edge_pass(g1b.reshape(NSC * N, D), g2b.reshape(NSC * N, H),
                          idxpb, zeros)
        h = _combine(ah, accf.reshape(NSC, NPAD, D), accb.reshape(NSC, NPAD, D),
                     last=(i == len(params) - 1))
    return h
